# Initial kernel scaffold; baseline (speedup 1.0000x reference)
import jax
import jax.numpy as jnp
from jax.experimental import pallas as pl


def kernel(x, edge_index, edge_weight, batch, Wl1, bl1, Wr1, Wl2, bl2, Wr2, p1, p2):
    raise NotImplementedError("write your pallas kernel here")


# speedup vs baseline: 5.6346x; 5.6346x over previous
import functools

import jax
import jax.numpy as jnp
from jax import lax
from jax.experimental import pallas as pl
from jax.experimental.pallas import tpu as pltpu
from jax.experimental.pallas import tpu_sc as plsc

NC = 2
NS = 16
NW = NC * NS

_N = 10000
_E = 320000
_D = 128
_K1 = 5000
_K2 = 2500
_NP1 = 10112


def _mesh():
    return plsc.VectorSubcoreMesh(core_axis_name="c", subcore_axis_name="s")


def _make_segsum(n_out, chunk, gather_flags):
    e_s = _E // NS
    n_iter = e_s // chunk
    stripe = n_out // NS

    @functools.partial(
        pl.kernel,
        mesh=_mesh(),
        out_type=[
            jax.ShapeDtypeStruct((n_out, _D), jnp.float32),
            jax.ShapeDtypeStruct((n_out, _D), jnp.float32),
        ],
        scratch_types=[
            pltpu.VMEM((chunk,), jnp.int32),
            pltpu.VMEM((chunk,), jnp.int32),
            pltpu.VMEM((chunk, _D), jnp.float32),
            pltpu.VMEM_SHARED((n_out, _D), jnp.float32),
            pltpu.SemaphoreType.DMA,
        ],
    )
    def seg(x_hbm, flag_hbm, src_hbm, dst_hbm, zblk_hbm, ones_hbm,
            agg_hbm, cnt_hbm, idx_s, idx_d, rows, shared, sem):
        cid = lax.axis_index("c")
        sid = lax.axis_index("s")
        pltpu.sync_copy(zblk_hbm, shared.at[pl.ds(sid * stripe, stripe)])
        plsc.subcore_barrier()

        base = sid * e_s

        def mkbody(table, do_gather):
            def body(i, carry):
                off = base + i * chunk
                if do_gather:
                    pltpu.sync_copy(src_hbm.at[pl.ds(off, chunk)], idx_s)
                pltpu.sync_copy(dst_hbm.at[pl.ds(off, chunk)], idx_d)
                if do_gather:
                    pltpu.async_copy(table.at[idx_s], rows, sem).wait()
                pltpu.sync_copy(rows, shared.at[idx_d], add=True)
                return carry
            return body

        @pl.when(cid == 0)
        def _():
            lax.fori_loop(0, n_iter, mkbody(x_hbm, True), 0)

        @pl.when(cid == 1)
        def _():
            if not gather_flags:
                pltpu.sync_copy(ones_hbm, rows)
            lax.fori_loop(0, n_iter, mkbody(flag_hbm, gather_flags), 0)

        plsc.subcore_barrier()

        @pl.when(cid == 0)
        def _():
            pltpu.sync_copy(shared.at[pl.ds(sid * stripe, stripe)],
                            agg_hbm.at[pl.ds(sid * stripe, stripe)])

        @pl.when(cid == 1)
        def _():
            pltpu.sync_copy(shared.at[pl.ds(sid * stripe, stripe)],
                            cnt_hbm.at[pl.ds(sid * stripe, stripe)])

    return seg


def _make_gather(b_total):
    b_w = b_total // NW
    sub = 80
    n_sub = b_w // sub

    @functools.partial(
        pl.kernel,
        mesh=_mesh(),
        out_type=jax.ShapeDtypeStruct((b_total, _D), jnp.float32),
        scratch_types=[
            pltpu.VMEM((sub,), jnp.int32),
            pltpu.VMEM((sub, _D), jnp.float32),
            pltpu.SemaphoreType.DMA,
        ],
    )
    def g(table_hbm, idx_hbm, out_hbm, idx_v, rows, sem):
        cid = lax.axis_index("c")
        sid = lax.axis_index("s")
        wid = sid * NC + cid
        base = wid * b_w
        for h in range(n_sub):
            off = base + h * sub
            pltpu.sync_copy(idx_hbm.at[pl.ds(off, sub)], idx_v)
            pltpu.async_copy(table_hbm.at[idx_v], rows, sem).wait()
            pltpu.sync_copy(rows, out_hbm.at[pl.ds(off, sub)])

    return g


def _dense_body(agg, cnt, x, Wl, bl, Wr, p, km, xo_ref, sc_ref):
    c = cnt[...][:, 0]
    mean = agg[...] / jnp.maximum(c, 1.0)[:, None]
    h = (jnp.dot(mean, Wl[...].T, preferred_element_type=jnp.float32)
         + bl[...][None, :]
         + jnp.dot(x[...], Wr[...].T, preferred_element_type=jnp.float32))
    xo = jnp.maximum(h, 0.0)
    xo_ref[...] = xo
    pv = p[...]
    ph = pv * lax.rsqrt(jnp.sum(pv * pv))
    s = jnp.tanh(jnp.sum(xo * ph[None, :], axis=1, keepdims=True))
    sc_ref[...] = jnp.where(km[...] > 0.0, s, jnp.float32(-3.0e38))


def _dense(agg, cnt, x, Wl, bl, Wr, p, km, n):
    return pl.pallas_call(
        _dense_body,
        out_shape=[
            jax.ShapeDtypeStruct((n, _D), jnp.float32),
            jax.ShapeDtypeStruct((n, 1), jnp.float32),
        ],
    )(agg, cnt, x, Wl, bl, Wr, p, km)


def _mscale_body(x1, vext, kmask, xp_ref, kf_ref):
    xp_ref[...] = x1[...] * vext[...]
    kf_ref[...] = kmask[...] * jnp.ones((1, _D), jnp.float32)


def _mscale(x1, vext, kmask):
    n = x1.shape[0]
    return pl.pallas_call(
        _mscale_body,
        out_shape=[
            jax.ShapeDtypeStruct((n, _D), jnp.float32),
            jax.ShapeDtypeStruct((n, _D), jnp.float32),
        ],
    )(x1, vext, kmask)


def _final_body(xp1, km, xg2, v2, o_ref):
    big_neg = jnp.float32(-3.0e38)
    x1 = xp1[...]
    mx1 = jnp.max(jnp.where(km[...] > 0.0, x1, big_neg), axis=0)
    mn1 = jnp.sum(x1, axis=0) / jnp.float32(_K1)
    xp2 = xg2[...] * v2[...]
    rid = lax.broadcasted_iota(jnp.int32, xp2.shape, 0)
    m = rid < _K2
    mx2 = jnp.max(jnp.where(m, xp2, big_neg), axis=0)
    mn2 = jnp.sum(jnp.where(m, xp2, 0.0), axis=0) / jnp.float32(_K2)
    o_ref[:, 0:_D] = mx1[None, :]
    o_ref[:, _D:2 * _D] = mn1[None, :]
    o_ref[:, 2 * _D:3 * _D] = mx2[None, :]
    o_ref[:, 3 * _D:4 * _D] = mn2[None, :]


def _final(xp1, km, xg2, v2):
    return pl.pallas_call(
        _final_body,
        out_shape=jax.ShapeDtypeStruct((1, 4 * _D), jnp.float32),
    )(xp1, km, xg2, v2)


def _pad_i32(a, n):
    return jnp.concatenate([a, jnp.zeros((n - a.shape[0],), jnp.int32)])


def kernel(x, edge_index, edge_weight, batch, Wl1, bl1, Wr1, Wl2, bl2, Wr2,
           p1, p2):
    src = edge_index[0]
    dst = edge_index[1]

    seg1 = _make_segsum(_NP1, 80, gather_flags=False)
    seg2 = _make_segsum(_NP1, 80, gather_flags=True)
    g2 = _make_gather(2560)

    zb = jnp.zeros((_NP1 // NS, _D), jnp.float32)
    ones_c = jnp.ones((80, _D), jnp.float32)
    ones_km = jnp.ones((_NP1, 1), jnp.float32)

    xpad = jnp.concatenate([x, jnp.zeros((_NP1 - _N, _D), jnp.float32)])

    agg1, cnt1 = seg1(xpad, xpad, src, dst, zb, ones_c)
    x1, s1 = _dense(agg1, cnt1, xpad, Wl1, bl1, Wr1, p1, ones_km, _NP1)

    vals1, perm1 = lax.top_k(s1[:_N, 0], _K1)
    vext = jnp.zeros((_NP1,), jnp.float32).at[perm1].set(vals1)
    kmask = jnp.zeros((_NP1,), jnp.float32).at[perm1].set(1.0)
    xp1e, kflag = _mscale(x1, vext[:, None], kmask[:, None])

    agg2, cnt2 = seg2(xp1e, kflag, src, dst, zb, ones_c)
    x2, s2 = _dense(agg2, cnt2, xp1e, Wl2, bl2, Wr2, p2, kmask[:, None],
                    _NP1)

    vals2, perm2 = lax.top_k(s2[:_N, 0], _K2)
    xg2 = g2(x2, _pad_i32(perm2, 2560))

    v2 = jnp.concatenate([vals2, jnp.zeros((60,), jnp.float32)])
    return _final(xp1e[:_N], kmask[:_N, None], xg2, v2[:, None])
